# baseline (speedup 1.0000x reference)
import functools

import jax
import jax.numpy as jnp
from jax import lax
from jax.experimental import pallas as pl
from jax.experimental.pallas import tpu as pltpu
from jax.experimental.pallas import tpu_sc as plsc

N_USERS = 5000
N_ITEMS = 5000
EMB = 128
N = N_USERS + N_ITEMS
E = 320000

NC = 2
NS = 16
NW = NC * NS
B = 128
NB0 = 160
NB_MAX = NB0
CH = 8
N_PAD = 10240

ROWS_PER_TILE = N_PAD // NS
ROW_CHUNK = 128
N_ROW_CHUNKS = ROWS_PER_TILE // ROW_CHUNK


def _sc_spmm_kernel(emb0_hbm, row_hbm, col_hbm, val_hbm, out_hbm,
                    acc, rowr, colr, valr, gbuf0, gbuf1,
                    gsem0, gsem1, ssem0, ssem1, wsem0, wsem1):
    c = lax.axis_index("c")
    s = lax.axis_index("s")
    w = s
    nb = NB0
    on_core0 = c == 0

    def issue_win(off, slot, sem):
        pltpu.async_copy(row_hbm.at[w, pl.ds(off, CH)], rowr.at[slot], sem)
        pltpu.async_copy(col_hbm.at[w, pl.ds(off, CH)], colr.at[slot], sem)
        pltpu.async_copy(val_hbm.at[w, pl.ds(off, CH)], valr.at[slot], sem)

    def wait_win(off, slot, sem):
        pltpu.make_async_copy(row_hbm.at[w, pl.ds(off, CH)], rowr.at[slot], sem).wait()
        pltpu.make_async_copy(col_hbm.at[w, pl.ds(off, CH)], colr.at[slot], sem).wait()
        pltpu.make_async_copy(val_hbm.at[w, pl.ds(off, CH)], valr.at[slot], sem).wait()

    base = s * ROWS_PER_TILE

    @pl.when(on_core0)
    def _():
        issue_win(0, 0, wsem0)
        issue_win(CH, 1, wsem1)

        def zrow(i, _):
            for r in range(8):
                gbuf0[i, pl.ds(r * 16, 16)] = jnp.zeros((16,), jnp.float32)
            return 0
        lax.fori_loop(0, B, zrow, 0)
        for j in range(N_ROW_CHUNKS):
            pltpu.async_copy(
                gbuf0, acc.at[pl.ds(base + j * ROW_CHUNK, ROW_CHUNK)], gsem1)
        for j in range(N_ROW_CHUNKS):
            pltpu.make_async_copy(
                gbuf0, acc.at[pl.ds(base + j * ROW_CHUNK, ROW_CHUNK)],
                gsem1).wait()

    plsc.subcore_barrier()

    def scale(buf, par, k):
        def body(g, _):
            vrow = valr[par, k, pl.ds(g * 16, 16)]
            for j in range(16):
                kk = g * 16 + j
                v = vrow[j]
                for r in range(8):
                    sl = pl.ds(r * 16, 16)
                    buf[kk, sl] = buf[kk, sl] * v
            return 0
        lax.fori_loop(0, B // 16, body, 0)

    def slot(b):
        return lax.rem(lax.div(b, CH), 2), lax.rem(b, CH)

    def batch_pair(g, _):
        b0 = 2 * g
        b1 = b0 + 1
        par, k0 = slot(b0)
        _, k1 = slot(b1)

        @pl.when(g > 0)
        def _():
            parp, kp = slot(b0 - 1)
            pltpu.make_async_copy(gbuf1, acc.at[rowr.at[parp, kp]], ssem1).wait()

        @pl.when((g > 0) & (lax.rem(g, CH // 2) == 0) & (b0 + CH < nb))
        def _():
            off = pl.multiple_of((lax.div(b0, CH) + 1) * CH, CH)

            @pl.when(par == 0)
            def _():
                issue_win(off, 1, wsem1)

            @pl.when(par == 1)
            def _():
                issue_win(off, 0, wsem0)

        pltpu.async_copy(emb0_hbm.at[colr.at[par, k1]], gbuf1, gsem1)

        pltpu.make_async_copy(emb0_hbm.at[colr.at[par, k0]], gbuf0, gsem0).wait()
        scale(gbuf0, par, k0)
        pltpu.async_copy(gbuf0, acc.at[rowr.at[par, k0]], ssem0, add=True)

        pltpu.make_async_copy(emb0_hbm.at[colr.at[par, k1]], gbuf1, gsem1).wait()
        scale(gbuf1, par, k1)
        pltpu.make_async_copy(gbuf0, acc.at[rowr.at[par, k0]], ssem0).wait()

        @pl.when(g < nb // 2 - 1)
        def _():
            parn, kn = slot(b0 + 2)

            @pl.when(lax.rem(b0 + 2, CH) == 0)
            def _():
                offn = pl.multiple_of(lax.div(b0 + 2, CH) * CH, CH)

                @pl.when(parn == 0)
                def _():
                    wait_win(offn, 0, wsem0)

                @pl.when(parn == 1)
                def _():
                    wait_win(offn, 1, wsem1)

            pltpu.async_copy(emb0_hbm.at[colr.at[parn, kn]], gbuf0, gsem0)
        pltpu.async_copy(gbuf1, acc.at[rowr.at[par, k1]], ssem1, add=True)
        return 0

    @pl.when(on_core0)
    def _():
        wait_win(0, 0, wsem0)
        pltpu.async_copy(emb0_hbm.at[colr.at[0, 0]], gbuf0, gsem0)
        lax.fori_loop(0, nb // 2, batch_pair, 0)
        parl, kl = slot(nb - 1)
        pltpu.make_async_copy(gbuf1, acc.at[rowr.at[parl, kl]], ssem1).wait()

    plsc.subcore_barrier()

    @pl.when(on_core0)
    def _():
        for j in range(N_ROW_CHUNKS):
            r0 = base + j * ROW_CHUNK
            pltpu.async_copy(
                acc.at[pl.ds(r0, ROW_CHUNK)], out_hbm.at[0, pl.ds(r0, ROW_CHUNK)],
                ssem0)
        for j in range(N_ROW_CHUNKS):
            r0 = base + j * ROW_CHUNK
            pltpu.make_async_copy(
                acc.at[pl.ds(r0, ROW_CHUNK)], out_hbm.at[0, pl.ds(r0, ROW_CHUNK)],
                ssem0).wait()


def _sc_spmm(emb0, rows, cols, vals):
    out_ref = jax.new_ref(jnp.zeros((1, N_PAD, EMB), jnp.float32))
    mesh = plsc.VectorSubcoreMesh(core_axis_name="c", subcore_axis_name="s")
    kfn = functools.partial(
        pl.kernel,
        mesh=mesh,
        out_type=(),
        scratch_types=[
            pltpu.VMEM_SHARED((N_PAD, EMB), jnp.float32),
            pltpu.VMEM((2, CH, B), jnp.int32),
            pltpu.VMEM((2, CH, B), jnp.int32),
            pltpu.VMEM((2, CH, B), jnp.float32),
            pltpu.VMEM((B, EMB), jnp.float32),
            pltpu.VMEM((B, EMB), jnp.float32),
            pltpu.SemaphoreType.DMA,
            pltpu.SemaphoreType.DMA,
            pltpu.SemaphoreType.DMA,
            pltpu.SemaphoreType.DMA,
            pltpu.SemaphoreType.DMA,
            pltpu.SemaphoreType.DMA,
        ],
    )(_sc_spmm_kernel)
    kfn(emb0, rows, cols, vals, out_ref)
    return out_ref[...]


def _tc_filter_kernel(emb0_ref, acc_ref, w_ref, out_ref):
    e = emb0_ref[...]
    spmm = acc_ref[0]
    x = 2.0 * e - spmm
    y = jax.nn.sigmoid(jnp.dot(x, w_ref[...], preferred_element_type=jnp.float32))
    out_ref[:, :EMB] = e
    out_ref[:, EMB:] = y


def _tc_filter(emb0, acc, filter_w):
    blk = 1024
    grid = N_PAD // blk
    return pl.pallas_call(
        _tc_filter_kernel,
        grid=(grid,),
        in_specs=[
            pl.BlockSpec((blk, EMB), lambda i: (i, 0)),
            pl.BlockSpec((1, blk, EMB), lambda i: (0, i, 0)),
            pl.BlockSpec((EMB, EMB), lambda i: (0, 0)),
        ],
        out_specs=pl.BlockSpec((blk, 2 * EMB), lambda i: (i, 0)),
        out_shape=jax.ShapeDtypeStruct((N_PAD, 2 * EMB), jnp.float32),
    )(emb0, acc, filter_w)


@jax.jit
def kernel(adj_indices, adj_values, user_embedding, item_embedding, filter_w):
    pad_rows = jnp.zeros((N_PAD - N, EMB), jnp.float32)
    emb0 = jnp.concatenate([user_embedding, item_embedding, pad_rows], axis=0)

    e_pad = NS * NB0 * B

    def _shard(a):
        return jnp.pad(a, (0, e_pad - E)).reshape(NS, NB0, B)

    row = _shard(adj_indices[0])
    col = _shard(adj_indices[1])
    val = _shard(adj_values)

    acc = _sc_spmm(emb0, row, col, val)
    all_emb = _tc_filter(emb0, acc, filter_w)
    return (all_emb[:N_USERS], all_emb[N_USERS:N])


# speedup vs baseline: 2.7918x; 2.7918x over previous
import functools

import jax
import jax.numpy as jnp
from jax import lax
from jax.experimental import pallas as pl
from jax.experimental.pallas import tpu as pltpu
from jax.experimental.pallas import tpu_sc as plsc

N_USERS = 5000
N_ITEMS = 5000
EMB = 128
N = N_USERS + N_ITEMS
E = 320000

NC = 2
NS = 16
NW = NC * NS
B = 128
NB0 = 80
NB_MAX = NB0
CH = 8
N_PAD = 10240

ROWS_PER_TILE = N_PAD // NS
ROW_CHUNK = 128
N_ROW_CHUNKS = ROWS_PER_TILE // ROW_CHUNK


def _sc_spmm_kernel(emb0_hbm, row_hbm, col_hbm, val_hbm, out_hbm,
                    acc, rowr, colr, valr, gbuf0, gbuf1,
                    gsem0, gsem1, ssem0, ssem1, wsem0, wsem1):
    c = lax.axis_index("c")
    s = lax.axis_index("s")
    w = c * NS + s
    nb = NB0

    def issue_win(off, slot, sem):
        pltpu.async_copy(row_hbm.at[w, pl.ds(off, CH)], rowr.at[slot], sem)
        pltpu.async_copy(col_hbm.at[w, pl.ds(off, CH)], colr.at[slot], sem)
        pltpu.async_copy(val_hbm.at[w, pl.ds(off, CH)], valr.at[slot], sem)

    def wait_win(off, slot, sem):
        pltpu.make_async_copy(row_hbm.at[w, pl.ds(off, CH)], rowr.at[slot], sem).wait()
        pltpu.make_async_copy(col_hbm.at[w, pl.ds(off, CH)], colr.at[slot], sem).wait()
        pltpu.make_async_copy(val_hbm.at[w, pl.ds(off, CH)], valr.at[slot], sem).wait()

    base = s * ROWS_PER_TILE

    issue_win(0, 0, wsem0)
    issue_win(CH, 1, wsem1)

    def zrow(i, _):
        for r in range(8):
            gbuf0[i, pl.ds(r * 16, 16)] = jnp.zeros((16,), jnp.float32)
        return 0
    lax.fori_loop(0, B, zrow, 0)
    for j in range(N_ROW_CHUNKS):
        pltpu.async_copy(
            gbuf0, acc.at[pl.ds(base + j * ROW_CHUNK, ROW_CHUNK)], gsem1)
    for j in range(N_ROW_CHUNKS):
        pltpu.make_async_copy(
            gbuf0, acc.at[pl.ds(base + j * ROW_CHUNK, ROW_CHUNK)],
            gsem1).wait()

    plsc.subcore_barrier()

    def scale(buf, par, k):
        def body(g, _):
            vrow = valr[par, k, pl.ds(g * 16, 16)]
            for j in range(16):
                kk = g * 16 + j
                v = vrow[j]
                for r in range(8):
                    sl = pl.ds(r * 16, 16)
                    buf[kk, sl] = buf[kk, sl] * v
            return 0
        lax.fori_loop(0, B // 16, body, 0)

    def slot(b):
        return lax.rem(lax.div(b, CH), 2), lax.rem(b, CH)

    def batch_pair(g, _):
        b0 = 2 * g
        b1 = b0 + 1
        par, k0 = slot(b0)
        _, k1 = slot(b1)

        @pl.when(g > 0)
        def _():
            parp, kp = slot(b0 - 1)
            pltpu.make_async_copy(gbuf1, acc.at[rowr.at[parp, kp]], ssem1).wait()

        @pl.when((g > 0) & (lax.rem(g, CH // 2) == 0) & (b0 + CH < nb))
        def _():
            off = pl.multiple_of((lax.div(b0, CH) + 1) * CH, CH)

            @pl.when(par == 0)
            def _():
                issue_win(off, 1, wsem1)

            @pl.when(par == 1)
            def _():
                issue_win(off, 0, wsem0)

        pltpu.async_copy(emb0_hbm.at[colr.at[par, k1]], gbuf1, gsem1)

        pltpu.make_async_copy(emb0_hbm.at[colr.at[par, k0]], gbuf0, gsem0).wait()
        scale(gbuf0, par, k0)
        pltpu.async_copy(gbuf0, acc.at[rowr.at[par, k0]], ssem0, add=True)

        pltpu.make_async_copy(emb0_hbm.at[colr.at[par, k1]], gbuf1, gsem1).wait()
        scale(gbuf1, par, k1)
        pltpu.make_async_copy(gbuf0, acc.at[rowr.at[par, k0]], ssem0).wait()

        @pl.when(g < nb // 2 - 1)
        def _():
            parn, kn = slot(b0 + 2)

            @pl.when(lax.rem(b0 + 2, CH) == 0)
            def _():
                offn = pl.multiple_of(lax.div(b0 + 2, CH) * CH, CH)

                @pl.when(parn == 0)
                def _():
                    wait_win(offn, 0, wsem0)

                @pl.when(parn == 1)
                def _():
                    wait_win(offn, 1, wsem1)

            pltpu.async_copy(emb0_hbm.at[colr.at[parn, kn]], gbuf0, gsem0)
        pltpu.async_copy(gbuf1, acc.at[rowr.at[par, k1]], ssem1, add=True)
        return 0

    wait_win(0, 0, wsem0)
    pltpu.async_copy(emb0_hbm.at[colr.at[0, 0]], gbuf0, gsem0)
    lax.fori_loop(0, nb // 2, batch_pair, 0)
    parl, kl = slot(nb - 1)
    pltpu.make_async_copy(gbuf1, acc.at[rowr.at[parl, kl]], ssem1).wait()

    plsc.subcore_barrier()

    for j in range(N_ROW_CHUNKS):
        r0 = base + j * ROW_CHUNK
        pltpu.async_copy(
            acc.at[pl.ds(r0, ROW_CHUNK)], out_hbm.at[c, pl.ds(r0, ROW_CHUNK)],
            ssem0)
    for j in range(N_ROW_CHUNKS):
        r0 = base + j * ROW_CHUNK
        pltpu.make_async_copy(
            acc.at[pl.ds(r0, ROW_CHUNK)], out_hbm.at[c, pl.ds(r0, ROW_CHUNK)],
            ssem0).wait()


def _sc_spmm(emb0, rows, cols, vals):
    out_ref = jax.new_ref(jnp.zeros((NC, N_PAD, EMB), jnp.float32))
    mesh = plsc.VectorSubcoreMesh(core_axis_name="c", subcore_axis_name="s")
    kfn = functools.partial(
        pl.kernel,
        mesh=mesh,
        out_type=(),
        scratch_types=[
            pltpu.VMEM_SHARED((N_PAD, EMB), jnp.float32),
            pltpu.VMEM((2, CH, B), jnp.int32),
            pltpu.VMEM((2, CH, B), jnp.int32),
            pltpu.VMEM((2, CH, B), jnp.float32),
            pltpu.VMEM((B, EMB), jnp.float32),
            pltpu.VMEM((B, EMB), jnp.float32),
            pltpu.SemaphoreType.DMA,
            pltpu.SemaphoreType.DMA,
            pltpu.SemaphoreType.DMA,
            pltpu.SemaphoreType.DMA,
            pltpu.SemaphoreType.DMA,
            pltpu.SemaphoreType.DMA,
        ],
    )(_sc_spmm_kernel)
    kfn(emb0, rows, cols, vals, out_ref)
    return out_ref[...]


def _tc_filter_kernel(emb0_ref, acc_ref, w_ref, out_ref):
    e = emb0_ref[...]
    spmm = acc_ref[0] + acc_ref[1]
    x = 2.0 * e - spmm
    y = jax.nn.sigmoid(jnp.dot(x, w_ref[...], preferred_element_type=jnp.float32))
    out_ref[:, :EMB] = e
    out_ref[:, EMB:] = y


def _tc_filter(emb0, acc, filter_w):
    blk = 1024
    grid = N_PAD // blk
    return pl.pallas_call(
        _tc_filter_kernel,
        grid=(grid,),
        in_specs=[
            pl.BlockSpec((blk, EMB), lambda i: (i, 0)),
            pl.BlockSpec((NC, blk, EMB), lambda i: (0, i, 0)),
            pl.BlockSpec((EMB, EMB), lambda i: (0, 0)),
        ],
        out_specs=pl.BlockSpec((blk, 2 * EMB), lambda i: (i, 0)),
        out_shape=jax.ShapeDtypeStruct((N_PAD, 2 * EMB), jnp.float32),
    )(emb0, acc, filter_w)


@jax.jit
def kernel(adj_indices, adj_values, user_embedding, item_embedding, filter_w):
    pad_rows = jnp.zeros((N_PAD - N, EMB), jnp.float32)
    emb0 = jnp.concatenate([user_embedding, item_embedding, pad_rows], axis=0)

    e_pad = NW * NB0 * B
    npad = e_pad - E
    spread = jnp.arange(npad, dtype=jnp.int32)
    row = jnp.concatenate(
        [adj_indices[0], N + spread % (N_PAD - N)]).reshape(NW, NB0, B)
    col = jnp.concatenate(
        [adj_indices[1], spread % N]).reshape(NW, NB0, B)
    val = jnp.concatenate(
        [adj_values, jnp.zeros((npad,), jnp.float32)]).reshape(NW, NB0, B)

    acc = _sc_spmm(emb0, row, col, val)
    all_emb = _tc_filter(emb0, acc, filter_w)
    return (all_emb[:N_USERS], all_emb[N_USERS:N])
